# baseline (speedup 1.0000x reference)
import functools

import jax
import jax.numpy as jnp
from jax import lax
from jax.experimental import pallas as pl
from jax.experimental.pallas import tpu as pltpu
from jax.experimental.pallas import tpu_sc as plsc

H = 1024
NH = 16
NKV = 4
HD = 64
E = 8
TOPK = 2
I_FF = 512
T = 2048
EPS = 1e-6
THETA = 10000.0
SCALE = HD ** -0.5

BT = 256
BQ = 256
MBT = 256
NPAIR = TOPK * T
P = NPAIR + E * MBT
NB = P // MBT
HALF = HD // 2


def _qkv_body(x_ref, w_ref, wqkv_ref, out_ref):
    x = x_ref[...]
    var = jnp.mean(x * x, axis=1, keepdims=True)
    xn = x * lax.rsqrt(var + EPS) * w_ref[...]
    out_ref[...] = jnp.dot(xn, wqkv_ref[...], preferred_element_type=jnp.float32)


def _attn_body(q_ref, k_ref, v_ref, cq_ref, sq_ref, ck_ref, sk_ref, o_ref):
    q2h = q_ref[...]
    cq = cq_ref[...]
    sq = sq_ref[...]
    h2 = pl.program_id(0)
    parity = (h2 // 2) % 2
    k128 = k_ref[...]
    v128 = v_ref[...]
    k = jnp.where(parity == 0, k128[:, :HD], k128[:, HD:])
    v = jnp.where(parity == 0, v128[:, :HD], v128[:, HD:])
    ck = ck_ref[...]
    sk = sk_ref[...]
    k1 = k[:, :HALF]
    k2 = k[:, HALF:]
    kr = jnp.concatenate([k1 * ck - k2 * sk, k2 * ck + k1 * sk], axis=1)
    qb = pl.program_id(1)
    row = qb * BQ + lax.broadcasted_iota(jnp.int32, (BQ, T), 0)
    col = lax.broadcasted_iota(jnp.int32, (BQ, T), 1)
    neg = jnp.float32(-1e9)

    def one_head(q):
        q1 = q[:, :HALF]
        q2 = q[:, HALF:]
        qr = jnp.concatenate([q1 * cq - q2 * sq, q2 * cq + q1 * sq], axis=1)
        sc = lax.dot_general(qr, kr, (((1,), (1,)), ((), ())),
                             preferred_element_type=jnp.float32) * SCALE
        sc = jnp.where(col <= row, sc, neg)
        m = jnp.max(sc, axis=1, keepdims=True)
        p = jnp.exp(sc - m)
        p = p / jnp.sum(p, axis=1, keepdims=True)
        return jnp.dot(p, v, preferred_element_type=jnp.float32)

    oa = one_head(q2h[:, :HD])
    ob = one_head(q2h[:, HD:])
    o_ref[...] = jnp.concatenate([oa, ob], axis=1)


def _post_body(ctx_ref, wd_ref, h_ref, ln2_ref, wg_ref, wsgu_ref, wsdn_ref,
               res_ref, x2_ref, ti_ref, tw_ref, sh_ref):
    attn = jnp.dot(ctx_ref[...], wd_ref[...], preferred_element_type=jnp.float32)
    res = attn + h_ref[...]
    res_ref[...] = res
    var = jnp.mean(res * res, axis=1, keepdims=True)
    x2 = res * lax.rsqrt(var + EPS) * ln2_ref[...]
    x2_ref[...] = x2

    logits = jnp.dot(x2, wg_ref[...], preferred_element_type=jnp.float32)
    col = lax.broadcasted_iota(jnp.int32, (BT, 128), 1)
    lm = jnp.where(col < E, logits, -1e30)
    m = jnp.max(lm, axis=1, keepdims=True)
    p = jnp.exp(lm - m)
    p = p / jnp.sum(p, axis=1, keepdims=True)
    v1 = jnp.max(p, axis=1, keepdims=True)
    idx1 = jnp.min(jnp.where(p == v1, col, 10000), axis=1, keepdims=True)
    p2 = jnp.where(col == idx1, -1.0, p)
    v2 = jnp.max(p2, axis=1, keepdims=True)
    idx2 = jnp.min(jnp.where(p2 == v2, col, 10000), axis=1, keepdims=True)
    denom = v1 + v2
    w1 = v1 / denom
    w2 = v2 / denom
    ti_ref[...] = jnp.where(col == 0, idx1, jnp.where(col == 1, idx2, 0)).astype(jnp.int32)
    tw_ref[...] = jnp.where(col == 0, w1, jnp.where(col == 1, w2, 0.0))

    gu = jnp.dot(x2.astype(jnp.bfloat16), wsgu_ref[...].astype(jnp.bfloat16),
                 preferred_element_type=jnp.float32)
    g = gu[:, :I_FF]
    u = gu[:, I_FF:]
    hsh = g * (1.0 / (1.0 + jnp.exp(-g))) * u
    sh_ref[...] = jnp.dot(hsh.astype(jnp.bfloat16),
                          wsdn_ref[...].astype(jnp.bfloat16),
                          preferred_element_type=jnp.float32)


MCH = 256


def _meta_body(e_ref, ps_ref, cnt_ref, r_scr):
    f32 = jnp.float32
    tri = (lax.broadcasted_iota(jnp.int32, (MCH, MCH), 0)
           >= lax.broadcasted_iota(jnp.int32, (MCH, MCH), 1)).astype(f32)
    lane = lax.broadcasted_iota(jnp.int32, (MCH, 128), 1)
    off = jnp.zeros((1, 128), f32)
    for c in range(NPAIR // MCH):
        e_c = e_ref[pl.ds(c * MCH, MCH), :]
        oh = (e_c == lane).astype(f32)
        cum = jnp.dot(tri, oh, preferred_element_type=f32) + off
        r_scr[pl.ds(c * MCH, MCH), :] = cum
        off = off + jnp.sum(oh, axis=0, keepdims=True)
    cnt_ref[...] = off
    padded = jnp.floor((off + (MBT - 1)) * (1.0 / MBT)).astype(jnp.int32).astype(f32) * MBT
    triL = (lax.broadcasted_iota(jnp.int32, (128, 128), 0)
            <= lax.broadcasted_iota(jnp.int32, (128, 128), 1)).astype(f32)
    pend = jnp.dot(padded, triL, preferred_element_type=f32)
    poff = pend - padded
    for c in range(NPAIR // MCH):
        e_c = e_ref[pl.ds(c * MCH, MCH), :]
        oh = (e_c == lane).astype(f32)
        cum = r_scr[pl.ds(c * MCH, MCH), :]
        vals = oh * (cum - 1.0 + poff)
        ps_ref[pl.ds(c * MCH, MCH), :] = jnp.sum(
            vals, axis=1, keepdims=True).astype(jnp.int32)


def _moe_body(be_ref, xs_ref, wgu_ref, wdn_ref, gw_ref, y_ref):
    del be_ref
    gu = jnp.dot(xs_ref[...].astype(jnp.bfloat16),
                 wgu_ref[0].astype(jnp.bfloat16),
                 preferred_element_type=jnp.float32)
    g = gu[:, :I_FF]
    u = gu[:, I_FF:]
    h = g * (1.0 / (1.0 + jnp.exp(-g))) * u
    y_ref[...] = jnp.dot(h.astype(jnp.bfloat16), wdn_ref[0].astype(jnp.bfloat16),
                         preferred_element_type=jnp.float32) * gw_ref[...]


_NW = 32
_TOKW = T // _NW
_CHT = 32


def _sc_combine(y_hbm, sh_hbm, p1_hbm, p2_hbm, out_hbm,
                i1_v, i2_v, y1_v, y2_v, sh_v, sem):
    wid = lax.axis_index("s") * 2 + lax.axis_index("c")
    base = wid * _TOKW

    def chunk(ci, carry):
        off = base + ci * _CHT
        pltpu.sync_copy(p1_hbm.at[pl.ds(off, _CHT)], i1_v)
        pltpu.sync_copy(p2_hbm.at[pl.ds(off, _CHT)], i2_v)
        pltpu.sync_copy(sh_hbm.at[pl.ds(off, _CHT)], sh_v)
        c1 = pltpu.async_copy(y_hbm.at[i1_v], y1_v, sem)
        c2 = pltpu.async_copy(y_hbm.at[i2_v], y2_v, sem)
        c1.wait()
        c2.wait()

        def tok(t, carry2):
            def col(j, carry3):
                sl = pl.ds(j * 16, 16)
                sh_v[t, sl] = sh_v[t, sl] + y1_v[t, sl] + y2_v[t, sl]
                return carry3

            return lax.fori_loop(0, H // 16, col, carry2)

        lax.fori_loop(0, _CHT, tok, 0)
        pltpu.sync_copy(sh_v, out_hbm.at[pl.ds(off, _CHT)])
        return carry

    lax.fori_loop(0, _TOKW // _CHT, chunk, 0)


def _combine_on_sc(y, shared, pos1, pos2):
    f32 = jnp.float32
    mesh = plsc.VectorSubcoreMesh(core_axis_name="c", subcore_axis_name="s")
    run = pl.kernel(
        _sc_combine,
        out_type=jax.ShapeDtypeStruct((T, H), f32),
        mesh=mesh,
        scratch_types=[
            pltpu.VMEM((_CHT,), jnp.int32),
            pltpu.VMEM((_CHT,), jnp.int32),
            pltpu.VMEM((_CHT, H), f32),
            pltpu.VMEM((_CHT, H), f32),
            pltpu.VMEM((_CHT, H), f32),
            pltpu.SemaphoreType.DMA,
        ],
    )
    return run(y, shared, pos1, pos2)


def kernel(hidden_states, position_ids, ln1_w, ln2_w, W_qkv, W_dense,
           W_gate, W_moe_gu, W_moe_down, W_sh_gu, W_sh_down):
    f32 = jnp.float32
    QKV_W = NH * HD + 2 * NKV * HD

    inv = 1.0 / (THETA ** (jnp.arange(0, HALF, dtype=f32) * 2.0 / HD))
    fr = position_ids.astype(f32)[:, None] * inv[None, :]
    cos = jnp.cos(fr)
    sin = jnp.sin(fr)

    ln1 = ln1_w.reshape(1, H)
    ln2 = ln2_w.reshape(1, H)

    qkv = pl.pallas_call(
        _qkv_body,
        grid=(T // BT,),
        in_specs=[
            pl.BlockSpec((BT, H), lambda b: (b, 0)),
            pl.BlockSpec((1, H), lambda b: (0, 0)),
            pl.BlockSpec((H, QKV_W), lambda b: (0, 0)),
        ],
        out_specs=pl.BlockSpec((BT, QKV_W), lambda b: (b, 0)),
        out_shape=jax.ShapeDtypeStruct((T, QKV_W), f32),
    )(hidden_states, ln1, W_qkv)

    grp = NH // NKV
    qh = qkv[:, :NH * HD].reshape(T, NH, HD).swapaxes(0, 1)
    ctx2 = pl.pallas_call(
        _attn_body,
        grid=(NH // 2, T // BQ),
        in_specs=[
            pl.BlockSpec((BQ, 2 * HD), lambda h2, qb: (qb, h2)),
            pl.BlockSpec((T, 2 * HD), lambda h2, qb: (0, NH // 2 + h2 // 4)),
            pl.BlockSpec((T, 2 * HD), lambda h2, qb: (0, (NH + NKV) // 2 + h2 // 4)),
            pl.BlockSpec((BQ, HALF), lambda h2, qb: (qb, 0)),
            pl.BlockSpec((BQ, HALF), lambda h2, qb: (qb, 0)),
            pl.BlockSpec((T, HALF), lambda h2, qb: (0, 0)),
            pl.BlockSpec((T, HALF), lambda h2, qb: (0, 0)),
        ],
        out_specs=pl.BlockSpec((BQ, 2 * HD), lambda h2, qb: (qb, h2)),
        out_shape=jax.ShapeDtypeStruct((T, NH * HD), f32),
    )(qkv, qkv, qkv, cos, sin, cos, sin)

    wg_pad = jnp.zeros((H, 128), f32).at[:, :E].set(W_gate)
    residual, x2, ti_pad, tw_pad, shared = pl.pallas_call(
        _post_body,
        grid=(T // BT,),
        in_specs=[
            pl.BlockSpec((BT, NH * HD), lambda b: (b, 0)),
            pl.BlockSpec((NH * HD, H), lambda b: (0, 0)),
            pl.BlockSpec((BT, H), lambda b: (b, 0)),
            pl.BlockSpec((1, H), lambda b: (0, 0)),
            pl.BlockSpec((H, 128), lambda b: (0, 0)),
            pl.BlockSpec((H, 2 * I_FF), lambda b: (0, 0)),
            pl.BlockSpec((I_FF, H), lambda b: (0, 0)),
        ],
        out_specs=[
            pl.BlockSpec((BT, H), lambda b: (b, 0)),
            pl.BlockSpec((BT, H), lambda b: (b, 0)),
            pl.BlockSpec((BT, 128), lambda b: (b, 0)),
            pl.BlockSpec((BT, 128), lambda b: (b, 0)),
            pl.BlockSpec((BT, H), lambda b: (b, 0)),
        ],
        out_shape=[
            jax.ShapeDtypeStruct((T, H), f32),
            jax.ShapeDtypeStruct((T, H), f32),
            jax.ShapeDtypeStruct((T, 128), jnp.int32),
            jax.ShapeDtypeStruct((T, 128), f32),
            jax.ShapeDtypeStruct((T, H), f32),
        ],
    )(ctx2, W_dense, hidden_states, ln2, wg_pad, W_sh_gu, W_sh_down)

    w1 = tw_pad[:, 0]
    w2 = tw_pad[:, 1]
    e2d = ti_pad[:, :TOPK].reshape(NPAIR, 1)
    ps2d, cnt = pl.pallas_call(
        _meta_body,
        grid=(1,),
        in_specs=[pl.BlockSpec((NPAIR, 1), lambda i: (0, 0))],
        out_specs=[
            pl.BlockSpec((NPAIR, 1), lambda i: (0, 0)),
            pl.BlockSpec((1, 128), lambda i: (0, 0)),
        ],
        out_shape=[
            jax.ShapeDtypeStruct((NPAIR, 1), jnp.int32),
            jax.ShapeDtypeStruct((1, 128), f32),
        ],
        scratch_shapes=[pltpu.VMEM((NPAIR, 128), f32)],
    )(e2d)
    ps = ps2d[:, 0]
    counts = cnt[0, :E].astype(jnp.int32)
    padded = ((counts + MBT - 1) // MBT) * MBT
    pend = jnp.cumsum(padded)
    src_tok = jnp.zeros((P,), jnp.int32).at[ps].set(
        (jnp.arange(NPAIR, dtype=jnp.int32) // TOPK))
    flat_w = tw_pad[:, :TOPK].reshape(-1)
    gw = jnp.zeros((P,), f32).at[ps].set(flat_w).reshape(P, 1)
    pos1 = ps[0::TOPK]
    pos2 = ps[1::TOPK]
    block_expert = jnp.minimum(
        jnp.searchsorted(pend, jnp.arange(NB, dtype=jnp.int32) * MBT,
                         side='right').astype(jnp.int32), E - 1)

    x_sorted = x2[src_tok]

    y = pl.pallas_call(
        _moe_body,
        grid_spec=pltpu.PrefetchScalarGridSpec(
            num_scalar_prefetch=1,
            grid=(NB,),
            in_specs=[
                pl.BlockSpec((MBT, H), lambda b, be: (b, 0)),
                pl.BlockSpec((1, H, 2 * I_FF), lambda b, be: (be[b], 0, 0)),
                pl.BlockSpec((1, I_FF, H), lambda b, be: (be[b], 0, 0)),
                pl.BlockSpec((MBT, 1), lambda b, be: (b, 0)),
            ],
            out_specs=pl.BlockSpec((MBT, H), lambda b, be: (b, 0)),
        ),
        out_shape=jax.ShapeDtypeStruct((P, H), f32),
    )(block_expert, x_sorted, W_moe_gu, W_moe_down, gw)

    mlp_out = _combine_on_sc(y, shared, pos1, pos2)

    return (mlp_out, residual)


# speedup vs baseline: 1.0247x; 1.0247x over previous
import functools

import jax
import jax.numpy as jnp
from jax import lax
from jax.experimental import pallas as pl
from jax.experimental.pallas import tpu as pltpu
from jax.experimental.pallas import tpu_sc as plsc

H = 1024
NH = 16
NKV = 4
HD = 64
E = 8
TOPK = 2
I_FF = 512
T = 2048
EPS = 1e-6
THETA = 10000.0
SCALE = HD ** -0.5

BT = 256
BQ = 256
MBT = 256
NPAIR = TOPK * T
P = NPAIR + E * MBT
NB = P // MBT
HALF = HD // 2


def _qkv_body(x_ref, w_ref, wqkv_ref, out_ref):
    x = x_ref[...]
    var = jnp.mean(x * x, axis=1, keepdims=True)
    xn = x * lax.rsqrt(var + EPS) * w_ref[...]
    out_ref[...] = jnp.dot(xn, wqkv_ref[...], preferred_element_type=jnp.float32)


def _attn_body(q_ref, k_ref, v_ref, cq_ref, sq_ref, ck_ref, sk_ref, o_ref):
    q2h = q_ref[...]
    cq = cq_ref[...]
    sq = sq_ref[...]
    h2 = pl.program_id(0)
    parity = (h2 // 2) % 2
    k128 = k_ref[...]
    v128 = v_ref[...]
    k = jnp.where(parity == 0, k128[:, :HD], k128[:, HD:])
    v = jnp.where(parity == 0, v128[:, :HD], v128[:, HD:])
    ck = ck_ref[...]
    sk = sk_ref[...]
    k1 = k[:, :HALF]
    k2 = k[:, HALF:]
    kr = jnp.concatenate([k1 * ck - k2 * sk, k2 * ck + k1 * sk], axis=1)
    qb = pl.program_id(1)
    row = qb * BQ + lax.broadcasted_iota(jnp.int32, (BQ, T), 0)
    col = lax.broadcasted_iota(jnp.int32, (BQ, T), 1)
    neg = jnp.float32(-1e9)

    def one_head(q):
        q1 = q[:, :HALF]
        q2 = q[:, HALF:]
        qr = jnp.concatenate([q1 * cq - q2 * sq, q2 * cq + q1 * sq], axis=1)
        sc = lax.dot_general(qr, kr, (((1,), (1,)), ((), ())),
                             preferred_element_type=jnp.float32) * SCALE
        sc = jnp.where(col <= row, sc, neg)
        m = jnp.max(sc, axis=1, keepdims=True)
        p = jnp.exp(sc - m)
        p = p / jnp.sum(p, axis=1, keepdims=True)
        return jnp.dot(p, v, preferred_element_type=jnp.float32)

    oa = one_head(q2h[:, :HD])
    ob = one_head(q2h[:, HD:])
    o_ref[...] = jnp.concatenate([oa, ob], axis=1)


def _post_body(ctx_ref, wd_ref, h_ref, ln2_ref, wg_ref, wsgu_ref, wsdn_ref,
               res_ref, x2_ref, ti_ref, tw_ref, sh_ref):
    attn = jnp.dot(ctx_ref[...], wd_ref[...], preferred_element_type=jnp.float32)
    res = attn + h_ref[...]
    res_ref[...] = res
    var = jnp.mean(res * res, axis=1, keepdims=True)
    x2 = res * lax.rsqrt(var + EPS) * ln2_ref[...]
    x2_ref[...] = x2

    logits = jnp.dot(x2, wg_ref[...], preferred_element_type=jnp.float32)
    col = lax.broadcasted_iota(jnp.int32, (BT, 128), 1)
    lm = jnp.where(col < E, logits, -1e30)
    m = jnp.max(lm, axis=1, keepdims=True)
    p = jnp.exp(lm - m)
    p = p / jnp.sum(p, axis=1, keepdims=True)
    v1 = jnp.max(p, axis=1, keepdims=True)
    idx1 = jnp.min(jnp.where(p == v1, col, 10000), axis=1, keepdims=True)
    p2 = jnp.where(col == idx1, -1.0, p)
    v2 = jnp.max(p2, axis=1, keepdims=True)
    idx2 = jnp.min(jnp.where(p2 == v2, col, 10000), axis=1, keepdims=True)
    denom = v1 + v2
    w1 = v1 / denom
    w2 = v2 / denom
    ti_ref[...] = jnp.where(col == 0, idx1, jnp.where(col == 1, idx2, 0)).astype(jnp.int32)
    tw_ref[...] = jnp.where(col == 0, w1, jnp.where(col == 1, w2, 0.0))

    gu = jnp.dot(x2.astype(jnp.bfloat16), wsgu_ref[...].astype(jnp.bfloat16),
                 preferred_element_type=jnp.float32)
    g = gu[:, :I_FF]
    u = gu[:, I_FF:]
    hsh = g * (1.0 / (1.0 + jnp.exp(-g))) * u
    sh_ref[...] = jnp.dot(hsh.astype(jnp.bfloat16),
                          wsdn_ref[...].astype(jnp.bfloat16),
                          preferred_element_type=jnp.float32)


MCH = 256


def _meta_body(e_ref, ps_ref, cnt_ref, r_scr):
    f32 = jnp.float32
    tri = (lax.broadcasted_iota(jnp.int32, (MCH, MCH), 0)
           >= lax.broadcasted_iota(jnp.int32, (MCH, MCH), 1)).astype(f32)
    lane = lax.broadcasted_iota(jnp.int32, (MCH, 128), 1)
    off = jnp.zeros((1, 128), f32)
    for c in range(NPAIR // MCH):
        e_c = e_ref[pl.ds(c * MCH, MCH), :]
        oh = (e_c == lane).astype(f32)
        cum = jnp.dot(tri, oh, preferred_element_type=f32) + off
        r_scr[pl.ds(c * MCH, MCH), :] = cum
        off = off + jnp.sum(oh, axis=0, keepdims=True)
    cnt_ref[...] = off
    padded = jnp.floor((off + (MBT - 1)) * (1.0 / MBT)).astype(jnp.int32).astype(f32) * MBT
    triL = (lax.broadcasted_iota(jnp.int32, (128, 128), 0)
            <= lax.broadcasted_iota(jnp.int32, (128, 128), 1)).astype(f32)
    pend = jnp.dot(padded, triL, preferred_element_type=f32)
    poff = pend - padded
    for c in range(NPAIR // MCH):
        e_c = e_ref[pl.ds(c * MCH, MCH), :]
        oh = (e_c == lane).astype(f32)
        cum = r_scr[pl.ds(c * MCH, MCH), :]
        vals = oh * (cum - 1.0 + poff)
        ps_ref[pl.ds(c * MCH, MCH), :] = jnp.sum(
            vals, axis=1, keepdims=True).astype(jnp.int32)


def _moe_body(be_ref, xs_ref, wgu_ref, wdn_ref, y_ref):
    del be_ref
    gu = jnp.dot(xs_ref[...].astype(jnp.bfloat16),
                 wgu_ref[0].astype(jnp.bfloat16),
                 preferred_element_type=jnp.float32)
    g = gu[:, :I_FF]
    u = gu[:, I_FF:]
    h = g * (1.0 / (1.0 + jnp.exp(-g))) * u
    y_ref[...] = jnp.dot(h.astype(jnp.bfloat16), wdn_ref[0].astype(jnp.bfloat16),
                         preferred_element_type=jnp.float32)


_NW = 32
_TOKW = T // _NW
_CHT = 32


def _sc_combine(y_hbm, sh_hbm, p1_hbm, p2_hbm, w1_hbm, w2_hbm, out_hbm,
                i1_v, i2_v, w1_v, w2_v, y1_v, y2_v, sh_v, sem):
    wid = lax.axis_index("s") * 2 + lax.axis_index("c")
    base = wid * _TOKW

    def chunk(ci, carry):
        off = base + ci * _CHT
        pltpu.sync_copy(p1_hbm.at[pl.ds(off, _CHT)], i1_v)
        pltpu.sync_copy(p2_hbm.at[pl.ds(off, _CHT)], i2_v)
        pltpu.sync_copy(w1_hbm.at[pl.ds(off, _CHT)], w1_v)
        pltpu.sync_copy(w2_hbm.at[pl.ds(off, _CHT)], w2_v)
        pltpu.sync_copy(sh_hbm.at[pl.ds(off, _CHT)], sh_v)
        c1 = pltpu.async_copy(y_hbm.at[i1_v], y1_v, sem)
        c2 = pltpu.async_copy(y_hbm.at[i2_v], y2_v, sem)
        c1.wait()
        c2.wait()

        def tok(t, carry2):
            w1v = w1_v[t, :]
            w2v = w2_v[t, :]

            def col(j, carry3):
                sl = pl.ds(j * 16, 16)
                sh_v[t, sl] = sh_v[t, sl] + w1v * y1_v[t, sl] + w2v * y2_v[t, sl]
                return carry3

            return lax.fori_loop(0, H // 16, col, carry2)

        lax.fori_loop(0, _CHT, tok, 0)
        pltpu.sync_copy(sh_v, out_hbm.at[pl.ds(off, _CHT)])
        return carry

    lax.fori_loop(0, _TOKW // _CHT, chunk, 0)


def _combine_on_sc(y, shared, pos1, pos2, w1, w2):
    f32 = jnp.float32
    mesh = plsc.VectorSubcoreMesh(core_axis_name="c", subcore_axis_name="s")
    run = pl.kernel(
        _sc_combine,
        out_type=jax.ShapeDtypeStruct((T, H), f32),
        mesh=mesh,
        scratch_types=[
            pltpu.VMEM((_CHT,), jnp.int32),
            pltpu.VMEM((_CHT,), jnp.int32),
            pltpu.VMEM((_CHT, 16), f32),
            pltpu.VMEM((_CHT, 16), f32),
            pltpu.VMEM((_CHT, H), f32),
            pltpu.VMEM((_CHT, H), f32),
            pltpu.VMEM((_CHT, H), f32),
            pltpu.SemaphoreType.DMA,
        ],
    )
    w1x = jnp.broadcast_to(w1[:, None], (T, 16))
    w2x = jnp.broadcast_to(w2[:, None], (T, 16))
    return run(y, shared, pos1, pos2, w1x, w2x)


def kernel(hidden_states, position_ids, ln1_w, ln2_w, W_qkv, W_dense,
           W_gate, W_moe_gu, W_moe_down, W_sh_gu, W_sh_down):
    f32 = jnp.float32
    QKV_W = NH * HD + 2 * NKV * HD

    inv = 1.0 / (THETA ** (jnp.arange(0, HALF, dtype=f32) * 2.0 / HD))
    fr = position_ids.astype(f32)[:, None] * inv[None, :]
    cos = jnp.cos(fr)
    sin = jnp.sin(fr)

    ln1 = ln1_w.reshape(1, H)
    ln2 = ln2_w.reshape(1, H)

    qkv = pl.pallas_call(
        _qkv_body,
        grid=(T // BT,),
        in_specs=[
            pl.BlockSpec((BT, H), lambda b: (b, 0)),
            pl.BlockSpec((1, H), lambda b: (0, 0)),
            pl.BlockSpec((H, QKV_W), lambda b: (0, 0)),
        ],
        out_specs=pl.BlockSpec((BT, QKV_W), lambda b: (b, 0)),
        out_shape=jax.ShapeDtypeStruct((T, QKV_W), f32),
    )(hidden_states, ln1, W_qkv)

    grp = NH // NKV
    qh = qkv[:, :NH * HD].reshape(T, NH, HD).swapaxes(0, 1)
    ctx2 = pl.pallas_call(
        _attn_body,
        grid=(NH // 2, T // BQ),
        in_specs=[
            pl.BlockSpec((BQ, 2 * HD), lambda h2, qb: (qb, h2)),
            pl.BlockSpec((T, 2 * HD), lambda h2, qb: (0, NH // 2 + h2 // 4)),
            pl.BlockSpec((T, 2 * HD), lambda h2, qb: (0, (NH + NKV) // 2 + h2 // 4)),
            pl.BlockSpec((BQ, HALF), lambda h2, qb: (qb, 0)),
            pl.BlockSpec((BQ, HALF), lambda h2, qb: (qb, 0)),
            pl.BlockSpec((T, HALF), lambda h2, qb: (0, 0)),
            pl.BlockSpec((T, HALF), lambda h2, qb: (0, 0)),
        ],
        out_specs=pl.BlockSpec((BQ, 2 * HD), lambda h2, qb: (qb, h2)),
        out_shape=jax.ShapeDtypeStruct((T, NH * HD), f32),
    )(qkv, qkv, qkv, cos, sin, cos, sin)

    wg_pad = jnp.zeros((H, 128), f32).at[:, :E].set(W_gate)
    residual, x2, ti_pad, tw_pad, shared = pl.pallas_call(
        _post_body,
        grid=(T // BT,),
        in_specs=[
            pl.BlockSpec((BT, NH * HD), lambda b: (b, 0)),
            pl.BlockSpec((NH * HD, H), lambda b: (0, 0)),
            pl.BlockSpec((BT, H), lambda b: (b, 0)),
            pl.BlockSpec((1, H), lambda b: (0, 0)),
            pl.BlockSpec((H, 128), lambda b: (0, 0)),
            pl.BlockSpec((H, 2 * I_FF), lambda b: (0, 0)),
            pl.BlockSpec((I_FF, H), lambda b: (0, 0)),
        ],
        out_specs=[
            pl.BlockSpec((BT, H), lambda b: (b, 0)),
            pl.BlockSpec((BT, H), lambda b: (b, 0)),
            pl.BlockSpec((BT, 128), lambda b: (b, 0)),
            pl.BlockSpec((BT, 128), lambda b: (b, 0)),
            pl.BlockSpec((BT, H), lambda b: (b, 0)),
        ],
        out_shape=[
            jax.ShapeDtypeStruct((T, H), f32),
            jax.ShapeDtypeStruct((T, H), f32),
            jax.ShapeDtypeStruct((T, 128), jnp.int32),
            jax.ShapeDtypeStruct((T, 128), f32),
            jax.ShapeDtypeStruct((T, H), f32),
        ],
    )(ctx2, W_dense, hidden_states, ln2, wg_pad, W_sh_gu, W_sh_down)

    w1 = tw_pad[:, 0]
    w2 = tw_pad[:, 1]
    e2d = ti_pad[:, :TOPK].reshape(NPAIR, 1)
    ps2d, cnt = pl.pallas_call(
        _meta_body,
        grid=(1,),
        in_specs=[pl.BlockSpec((NPAIR, 1), lambda i: (0, 0))],
        out_specs=[
            pl.BlockSpec((NPAIR, 1), lambda i: (0, 0)),
            pl.BlockSpec((1, 128), lambda i: (0, 0)),
        ],
        out_shape=[
            jax.ShapeDtypeStruct((NPAIR, 1), jnp.int32),
            jax.ShapeDtypeStruct((1, 128), f32),
        ],
        scratch_shapes=[pltpu.VMEM((NPAIR, 128), f32)],
    )(e2d)
    ps = ps2d[:, 0]
    counts = cnt[0, :E].astype(jnp.int32)
    padded = ((counts + MBT - 1) // MBT) * MBT
    pend = jnp.cumsum(padded)
    src_tok = jnp.zeros((P,), jnp.int32).at[ps].set(
        (jnp.arange(NPAIR, dtype=jnp.int32) // TOPK))
    pos1 = ps[0::TOPK]
    pos2 = ps[1::TOPK]
    block_expert = jnp.minimum(
        jnp.searchsorted(pend, jnp.arange(NB, dtype=jnp.int32) * MBT,
                         side='right').astype(jnp.int32), E - 1)

    x_sorted = x2[src_tok]

    y = pl.pallas_call(
        _moe_body,
        grid_spec=pltpu.PrefetchScalarGridSpec(
            num_scalar_prefetch=1,
            grid=(NB,),
            in_specs=[
                pl.BlockSpec((MBT, H), lambda b, be: (b, 0)),
                pl.BlockSpec((1, H, 2 * I_FF), lambda b, be: (be[b], 0, 0)),
                pl.BlockSpec((1, I_FF, H), lambda b, be: (be[b], 0, 0)),
            ],
            out_specs=pl.BlockSpec((MBT, H), lambda b, be: (b, 0)),
        ),
        out_shape=jax.ShapeDtypeStruct((P, H), f32),
    )(block_expert, x_sorted, W_moe_gu, W_moe_down)

    mlp_out = _combine_on_sc(y, shared, pos1, pos2, w1, w2)

    return (mlp_out, residual)
